# baseline (speedup 1.0000x reference)
import jax
import jax.numpy as jnp
from jax import lax
from jax.experimental import pallas as pl
from jax.experimental.pallas import tpu as pltpu
from jax.experimental.pallas import tpu_sc as plsc

_D = 768
_LT = _D // 128
_NW = 32
_C = 64
_G = _C // 8
_N_CHUNKS = 20 * 4096 // (_NW * _C)


def _emb_body(idx_hbm, table_hbm, out_hbm, idx_v, buf0, buf1, gs0, gs1, ss0, ss1):
    bufs = (buf0, buf1)
    gsems = (gs0, gs1)
    ssems = (ss0, ss1)
    wid = lax.axis_index("s") * 2 + lax.axis_index("c")
    pltpu.sync_copy(idx_hbm.at[wid], idx_v)

    def gathers(c, b, issue):
        for g in range(_G):
            d = pltpu.make_async_copy(
                table_hbm.at[idx_v.at[c, pl.ds(g * 8, 8)]], bufs[b].at[g], gsems[b]
            )
            d.start() if issue else d.wait()

    def stores(c, b, issue):
        for t in range(_LT):
            d = pltpu.make_async_copy(
                bufs[b].at[:, :, pl.ds(t * 128, 128)],
                out_hbm.at[wid, c, :, t],
                ssems[b],
            )
            d.start() if issue else d.wait()

    gathers(0, 0, True)

    @pl.loop(0, _N_CHUNKS, step=2)
    def _(j):
        for b in range(2):
            c = j + b
            bn = 1 - b

            @pl.when(c + 1 < _N_CHUNKS)
            def _():
                @pl.when(c >= 1)
                def _():
                    stores(c - 1, bn, False)
                gathers(c + 1, bn, True)

            gathers(c, b, False)
            stores(c, b, True)

    stores(_N_CHUNKS - 2, (_N_CHUNKS - 2) % 2, False)
    stores(_N_CHUNKS - 1, (_N_CHUNKS - 1) % 2, False)


@jax.jit
def _emb(idx, weight):
    mesh = plsc.VectorSubcoreMesh(core_axis_name="c", subcore_axis_name="s")
    return pl.kernel(
        _emb_body,
        out_type=jax.ShapeDtypeStruct((_NW, _N_CHUNKS, _G, _LT, 8, 128), jnp.float32),
        mesh=mesh,
        scratch_types=(
            [pltpu.VMEM((_N_CHUNKS, _C), jnp.int32)]
            + [pltpu.VMEM((_G, 8, _D), jnp.float32) for _ in range(2)]
            + [pltpu.SemaphoreType.DMA for _ in range(4)]
        ),
    )(idx, weight)


def kernel(text, weight):
    b, h = text.shape
    idx = text.T.reshape(_NW, _N_CHUNKS, _C).astype(jnp.int32)
    out = _emb(idx, weight)
    x = out.reshape(h * b // 8, _LT, 8, 128).transpose(0, 2, 1, 3)
    return x.reshape(h, b, _D).transpose(1, 0, 2)
